# baseline (speedup 1.0000x reference)
import functools

import jax
import jax.numpy as jnp
from jax import lax
from jax.experimental import pallas as pl
from jax.experimental.pallas import tpu as pltpu
from jax.experimental.pallas import tpu_sc as plsc

_B, _N, _T = 8, 8192, 128
_K = 1024
_STRIDE = 4
_T1, _T2, _CHN, _KS = 31, 6, 8, 8
_CAND = 2048

_NC, _NS = 2, 16
_NW = _NC * _NS
_CH = 128


def _expand_weights(W0, b0, W1, b1):
    S1 = 32
    t1 = jnp.arange(_T1)
    c = jnp.arange(_CHN)
    k = jnp.arange(_KS)
    rows1 = jnp.broadcast_to(_STRIDE * t1[:, None, None] + k[None, None, :],
                             (_T1, _CHN, _KS)).reshape(-1)
    cols1 = jnp.broadcast_to(c[None, :, None] * S1 + t1[:, None, None],
                             (_T1, _CHN, _KS)).reshape(-1)
    M1 = jnp.zeros((_T, S1 * _CHN), jnp.float32)
    M1 = M1.at[rows1, cols1].set(
        jnp.broadcast_to(W0[:, 0, :][None, :, :], (_T1, _CHN, _KS)).reshape(-1))
    B1 = jnp.zeros((S1 * _CHN,), jnp.float32)
    B1 = B1.at[cols1].set(jnp.broadcast_to(b0[None, :, None], (_T1, _CHN, _KS)).reshape(-1))
    t2 = jnp.arange(_T2)
    ci = jnp.arange(_CHN)
    rows2 = jnp.broadcast_to(ci[None, :, None, None] * S1 + _STRIDE * t2[:, None, None, None]
                             + k[None, None, None, :], (_T2, _CHN, _CHN, _KS)).reshape(-1)
    cols2 = jnp.broadcast_to(c[None, None, :, None] * _T2 + t2[:, None, None, None],
                             (_T2, _CHN, _CHN, _KS)).reshape(-1)
    M2 = jnp.zeros((S1 * _CHN, _T2 * _CHN), jnp.float32)
    vals2 = jnp.transpose(W1, (1, 0, 2))[None]
    M2 = M2.at[rows2, cols2].set(
        jnp.broadcast_to(vals2, (_T2, _CHN, _CHN, _KS)).reshape(-1))
    B2 = jnp.repeat(b1, _T2)
    return M1, B1, M2, B2


def _fast_scores(h, W0, b0, W1, b1, Wp, bp):
    M1, B1, M2, B2 = _expand_weights(W0, b0, W1, b1)
    x = h.reshape(_B * _N, _T)
    o1 = jnp.maximum(x @ M1 + B1, 0.0)
    z = jnp.maximum(o1 @ M2 + B2, 0.0)
    weights = z @ Wp.T + bp
    return jax.nn.sigmoid(weights.reshape(_B, _N))


def _exact_scores(hc, W0, b0, W1, b1, Wp, bp):
    n = hc.shape[0]
    x = hc.reshape(n, 1, _T)
    for W, b in ((W0, b0), (W1, b1)):
        x = lax.conv_general_dilated(x, W, window_strides=(_STRIDE,), padding='VALID',
                                     dimension_numbers=('NCH', 'OIH', 'NCH'))
        x = jnp.maximum(x + b[None, :, None], 0.0)
    z = x.reshape(_B, n // _B, -1)
    weights = z @ Wp.T + bp
    return jax.nn.sigmoid(weights[..., 0])


def _cmpx(v, i, d, desc, e_sub, e_lane):
    if d < 128:
        ov_dn, ov_up = jnp.roll(v, -d, axis=-1), jnp.roll(v, d, axis=-1)
        oi_dn, oi_up = jnp.roll(i, -d, axis=-1), jnp.roll(i, d, axis=-1)
        lower = (e_lane & d) == 0
    else:
        sd = d // 128
        ov_dn, ov_up = jnp.roll(v, -sd, axis=-2), jnp.roll(v, sd, axis=-2)
        oi_dn, oi_up = jnp.roll(i, -sd, axis=-2), jnp.roll(i, sd, axis=-2)
        lower = (e_sub & sd) == 0
    ov = jnp.where(lower, ov_dn, ov_up)
    oi = jnp.where(lower, oi_dn, oi_up)
    selfwins = (v > ov) | ((v == ov) & (i < oi))
    c = selfwins == (lower == desc)
    return jnp.where(c, v, ov), jnp.where(c, i, oi)


def _topk_body(nrounds, s_ref, ii_ref, v_ref, i_ref):
    v = s_ref[...]
    i = ii_ref[...]
    rows = v.shape[0]
    e_sub = lax.broadcasted_iota(jnp.int32, (1, 8, 128), 1)
    e_lane = lax.broadcasted_iota(jnp.int32, (1, 8, 128), 2)
    e = e_sub * 128 + e_lane
    row = lax.broadcasted_iota(jnp.int32, (rows, 1, 1), 0)
    row_asc = (row % 2) == 1

    for k in range(1, 11):
        for j in reversed(range(k)):
            d = 1 << j
            if k < 10:
                desc_blk = ((e >> k) & 1) == 0
            else:
                desc_blk = jnp.full((1, 8, 128), True)
            v, i = _cmpx(v, i, d, desc_blk != row_asc, e_sub, e_lane)

    for r in range(nrounds):
        rows = v.shape[0] // 2
        v2 = v.reshape(rows, 2, 8, 128)
        i2 = i.reshape(rows, 2, 8, 128)
        va, vb = v2[:, 0], v2[:, 1]
        ia, ib = i2[:, 0], i2[:, 1]
        awins = (va > vb) | ((va == vb) & (ia < ib))
        v = jnp.where(awins, va, vb)
        i = jnp.where(awins, ia, ib)
        row = lax.broadcasted_iota(jnp.int32, (rows, 1, 1), 0)
        row_asc = ((row % 2) == 1) if r < nrounds - 1 else jnp.full((rows, 1, 1), False)
        desc = jnp.broadcast_to(jnp.logical_not(row_asc), (rows, 8, 128))
        for j in reversed(range(10)):
            v, i = _cmpx(v, i, 1 << j, desc, e_sub, e_lane)

    v_ref[...] = v
    i_ref[...] = i


def _topk_pallas(scores, idx, nrounds):
    rows = scores.shape[0] * scores.shape[1] // 1024
    orows = rows >> nrounds
    s4 = scores.reshape(rows, 8, 128)
    i4 = idx.reshape(rows, 8, 128)
    vals4, idx4 = pl.pallas_call(
        functools.partial(_topk_body, nrounds),
        in_specs=[pl.BlockSpec((rows, 8, 128), lambda: (0, 0, 0)),
                  pl.BlockSpec((rows, 8, 128), lambda: (0, 0, 0))],
        out_specs=[pl.BlockSpec((orows, 8, 128), lambda: (0, 0, 0)),
                   pl.BlockSpec((orows, 8, 128), lambda: (0, 0, 0))],
        out_shape=[jax.ShapeDtypeStruct((orows, 8, 128), jnp.float32),
                   jax.ShapeDtypeStruct((orows, 8, 128), jnp.int32)],
    )(s4, i4)
    n_out = orows * 1024 // _B
    return vals4.reshape(_B, n_out), idx4.reshape(_B, n_out)


def _sc_gather_body(nchunk, table_hbm, idx_hbm, out_hbm, idx_v, rows_v, sem):
    wid = lax.axis_index("s") * _NC + lax.axis_index("c")
    base = wid * (nchunk * _CH)
    cps = []
    for j in range(nchunk):
        pltpu.sync_copy(idx_hbm.at[pl.ds(base + j * _CH, _CH)], idx_v[j])
        cps.append(pltpu.async_copy(table_hbm.at[idx_v[j]], rows_v[j], sem))
    for j in range(nchunk):
        cps[j].wait()
        pltpu.sync_copy(rows_v[j], out_hbm.at[pl.ds(base + j * _CH, _CH)])


def _sc_gather(table, flat_idx):
    nrows = flat_idx.shape[0]
    nchunk = nrows // (_NW * _CH)
    mesh = plsc.VectorSubcoreMesh(core_axis_name="c", subcore_axis_name="s")
    scratch = ([pltpu.VMEM((_CH,), jnp.int32) for _ in range(nchunk)]
               + [pltpu.VMEM((_CH, _T), jnp.float32) for _ in range(nchunk)]
               + [pltpu.SemaphoreType.DMA])

    def body(table_hbm, idx_hbm, out_hbm, *refs):
        idx_v = refs[:nchunk]
        rows_v = refs[nchunk:2 * nchunk]
        sem = refs[-1]
        _sc_gather_body(nchunk, table_hbm, idx_hbm, out_hbm, idx_v, rows_v, sem)

    fn = functools.partial(
        pl.kernel, mesh=mesh,
        out_type=jax.ShapeDtypeStruct((nrows, _T), jnp.float32),
        scratch_types=scratch,
    )(body)
    return fn(table, flat_idx)


def _scale_body(g_ref, v_ref, o_ref):
    v = v_ref[0, 0, :]
    o_ref[...] = g_ref[...] * v[None, :, None]


def kernel(h, W0, b0, W1, b1, Wp, bp):
    s_fast = _fast_scores(h, W0, b0, W1, b1, Wp, bp)
    elem = jnp.broadcast_to(
        (jnp.arange(_N, dtype=jnp.int32))[None, :], (_B, _N))
    _, cand_idx = _topk_pallas(s_fast, elem, nrounds=2)
    cand_flat = (cand_idx + jnp.arange(_B, dtype=jnp.int32)[:, None] * _N).reshape(-1)
    hc = _sc_gather(h.reshape(_B * _N, _T), cand_flat)
    s_exact = _exact_scores(hc, W0, b0, W1, b1, Wp, bp)
    vals, idx = _topk_pallas(s_exact, cand_idx, nrounds=1)
    flat_idx = (idx + jnp.arange(_B, dtype=jnp.int32)[:, None] * _N).reshape(-1)
    g = _sc_gather(h.reshape(_B * _N, _T), flat_idx).reshape(_B, _K, _T)
    new_h = pl.pallas_call(
        _scale_body,
        grid=(_B,),
        in_specs=[
            pl.BlockSpec((1, _K, _T), lambda b: (b, 0, 0)),
            pl.BlockSpec((1, 1, _K), lambda b: (b, 0, 0)),
        ],
        out_specs=pl.BlockSpec((1, _K, _T), lambda b: (b, 0, 0)),
        out_shape=jax.ShapeDtypeStruct((_B, _K, _T), jnp.float32),
    )(g, vals[:, None, :])
    return new_h, idx[:, :, None]
